# Initial kernel scaffold; baseline (speedup 1.0000x reference)
import jax
import jax.numpy as jnp
from jax.experimental import pallas as pl


def kernel(states, site_flat_idx):
    raise NotImplementedError("write your pallas kernel here")


# speedup vs baseline: 1.8942x; 1.8942x over previous
import functools

import jax
import jax.numpy as jnp
from jax import lax
from jax.experimental import pallas as pl
from jax.experimental.pallas import tpu as pltpu
from jax.experimental.pallas import tpu_sc as plsc

_A = 256
_S = 32768
_M = 501 * 501 * 2 * 2
_NW = 32
_R = 31376
_M_PAD = _NW * _R
_SROW = 256
_BLK = 32


def _field_body(idx_ref, st_ref, out_ref):
    idx = idx_ref[...]
    i0 = idx // 2004
    rem = idx - i0 * 2004
    i1 = rem // 4
    r4 = rem - i1 * 4
    i2 = r4 // 2
    i3 = r4 - i2 * 2
    xs = i0.astype(jnp.float32) - 250.0
    ys = i1.astype(jnp.float32) - 250.0
    ss = i2.astype(jnp.float32) - 0.5
    ts = i3.astype(jnp.float32) - 0.5
    tq = ts + 0.5

    def body(i, carry):
        acc_q, acc_p = carry
        xi = st_ref[i, 0]
        yi = st_ref[i, 1]
        si = st_ref[i, 2]
        ti = st_ref[i, 3]
        dx = xs - xi
        dy = ys - yi
        dist = jnp.sqrt(dx * dx + dy * dy + 1e-12)
        acc_q = acc_q + (ti + 0.5) / (dist + 1.0)
        sd = dist + jnp.abs(ss - si) + jnp.abs(ts - ti)
        acc_p = acc_p + jnp.where(sd < 1e-3, 1e6, 0.0)
        return acc_q, acc_p

    z = jnp.zeros_like(xs)
    acc_q, acc_p = lax.fori_loop(0, _A, body, (z, z))
    out_ref[...] = acc_p + tq * acc_q


def _compute_field(idx2d, states):
    return pl.pallas_call(
        _field_body,
        grid=(_SROW // _BLK,),
        in_specs=[
            pl.BlockSpec((_BLK, 128), lambda i: (i, 0)),
            pl.BlockSpec(memory_space=pltpu.SMEM),
        ],
        out_specs=pl.BlockSpec((_BLK, 128), lambda i: (i, 0)),
        out_shape=jax.ShapeDtypeStruct((_SROW, 128), jnp.float32),
    )(idx2d, states)


def _sc_scatter(field, idx):
    mesh = plsc.VectorSubcoreMesh(core_axis_name="c", subcore_axis_name="s")

    @functools.partial(
        pl.kernel,
        mesh=mesh,
        out_type=jax.ShapeDtypeStruct((_M_PAD,), jnp.float32),
        scratch_types=[
            pltpu.VMEM((_R,), jnp.float32),
            pltpu.VMEM((_S,), jnp.int32),
            pltpu.VMEM((_S,), jnp.float32),
        ],
        compiler_params=pltpu.CompilerParams(needs_layout_passes=False),
    )
    def k(field_hbm, idx_hbm, out_hbm, buf, idx_v, val_v):
        wid = lax.axis_index("s") * 2 + lax.axis_index("c")
        base = pl.multiple_of(wid * _R, 8)
        pltpu.sync_copy(idx_hbm, idx_v)
        pltpu.sync_copy(field_hbm, val_v)

        def zero(j, c):
            buf[pl.ds(j * 16, 16)] = jnp.zeros((16,), jnp.float32)
            return c

        lax.fori_loop(0, _R // 16, zero, 0)

        def scat(g, c):
            iv = idx_v[pl.ds(g * 16, 16)]
            rel = iv - base
            m = (rel >= 0) & (rel < _R)
            relc = jnp.where(m, rel, 0)
            vals = val_v[pl.ds(g * 16, 16)]
            plsc.store_scatter(buf, [relc], vals, mask=m)
            return c

        lax.fori_loop(0, _S // 16, scat, 0)

        pltpu.sync_copy(buf, out_hbm.at[pl.ds(base, _R)])

    return k(field, idx)


def kernel(states, site_flat_idx):
    idx2d = site_flat_idx.reshape(_SROW, 128)
    field = _compute_field(idx2d, states).reshape(_S)
    out = _sc_scatter(field, site_flat_idx)
    return out[:_M]
